# baseline (speedup 1.0000x reference)
import functools

import jax
import jax.numpy as jnp
from jax import lax
from jax.experimental import pallas as pl
from jax.experimental.pallas import tpu as pltpu
from jax.experimental.pallas import tpu_sc as plsc

_BLK_S = 1024
_NBUF = 3

_SC_NC = 2
_SC_NS = 16
_SC_NW = _SC_NC * _SC_NS
_SC_LANES = 16
_SC_CW = 16384


def _add_block(x_ref, t_ref, o_ref):
    o_ref[...] = x_ref[...] + t_ref[...]


def _tc_add(x, table, nb):
    b, s, d = x.shape

    def outer(x_hbm, t_hbm, o_hbm):
        pipeline = pltpu.emit_pipeline(
            _add_block,
            grid=(s // _BLK_S, nb),
            in_specs=[
                pl.BlockSpec((1, _BLK_S, d), lambda j, i: (i, j, 0),
                             pipeline_mode=pl.Buffered(buffer_count=_NBUF)),
                pl.BlockSpec((_BLK_S, d), lambda j, i: (j, 0)),
            ],
            out_specs=[
                pl.BlockSpec((1, _BLK_S, d), lambda j, i: (i, j, 0)),
            ],
        )
        pipeline(x_hbm, t_hbm, o_hbm)

    return pl.pallas_call(
        outer,
        in_specs=[
            pl.BlockSpec(memory_space=pl.ANY),
            pl.BlockSpec(memory_space=pl.ANY),
        ],
        out_specs=pl.BlockSpec(memory_space=pl.ANY),
        out_shape=jax.ShapeDtypeStruct((nb, s, d), x.dtype),
    )(x, table)


def _sc_add(x_flat, t_flat, x_off, n):
    words_per_w = n // _SC_NW
    nch = words_per_w // _SC_CW
    mesh = plsc.VectorSubcoreMesh(core_axis_name="c", subcore_axis_name="s")

    @functools.partial(
        pl.kernel,
        out_type=jax.ShapeDtypeStruct((n,), jnp.float32),
        mesh=mesh,
        scratch_types=[
            pltpu.VMEM((_SC_CW,), jnp.float32),
            pltpu.VMEM((_SC_CW,), jnp.float32),
            pltpu.VMEM((_SC_CW,), jnp.float32),
            pltpu.VMEM((_SC_CW,), jnp.float32),
            pltpu.SemaphoreType.DMA,
            pltpu.SemaphoreType.DMA,
            pltpu.SemaphoreType.DMA,
            pltpu.SemaphoreType.DMA,
            pltpu.SemaphoreType.DMA,
            pltpu.SemaphoreType.DMA,
        ],
    )
    def _sc_kernel(x_hbm, t_hbm, o_hbm, bx0, bx1, bt0, bt1,
                   sx0, sx1, st0, st1, so0, so1):
        wid = lax.axis_index("s") * _SC_NC + lax.axis_index("c")
        tb = wid * words_per_w
        bx = (bx0, bx1)
        bt = (bt0, bt1)
        sx = (sx0, sx1)
        st = (st0, st1)
        so = (so0, so1)
        inx = [None, None]
        int_ = [None, None]
        outd = [None, None]

        def start_in(c):
            sl = c & 1
            off = tb + c * _SC_CW
            inx[sl] = pltpu.async_copy(
                x_hbm.at[pl.ds(x_off + off, _SC_CW)], bx[sl], sx[sl])
            int_[sl] = pltpu.async_copy(
                t_hbm.at[pl.ds(off, _SC_CW)], bt[sl], st[sl])

        start_in(0)
        for c in range(nch):
            sl = c & 1
            inx[sl].wait()
            int_[sl].wait()
            bxs, bts = bx[sl], bt[sl]

            @plsc.parallel_loop(0, _SC_CW, step=_SC_LANES, unroll=8)
            def _(o):
                s2 = pl.ds(o, _SC_LANES)
                plsc.addupdate(bxs.at[s2], bts[s2])

            outd[sl] = pltpu.async_copy(
                bxs, o_hbm.at[pl.ds(tb + c * _SC_CW, _SC_CW)], so[sl])
            if c + 1 < nch:
                nsl = (c + 1) & 1
                if outd[nsl] is not None:
                    outd[nsl].wait()
                start_in(c + 1)
        outd[0].wait()
        outd[1].wait()

    return _sc_kernel(x_flat, t_flat)


def kernel(x, table):
    b, s, d = x.shape
    table_s = table[:s]
    out_tc = _tc_add(x, table_s, b - 1)
    out_sc = _sc_add(x.reshape(-1), table_s.reshape(-1),
                     (b - 1) * s * d, s * d)
    return jnp.concatenate([out_tc, out_sc.reshape(1, s, d)], axis=0)


# speedup vs baseline: 4.2958x; 4.2958x over previous
import jax
import jax.numpy as jnp
from jax.experimental import pallas as pl
from jax.experimental.pallas import tpu as pltpu

_BLK_S = 1024
_NBUF = 3


def _add_block(x_ref, t_ref, o_ref):
    o_ref[...] = x_ref[...] + t_ref[...]


def kernel(x, table):
    b, s, d = x.shape
    table_s = table[:s]

    def outer(x_hbm, t_hbm, o_hbm):
        pipeline = pltpu.emit_pipeline(
            _add_block,
            grid=(s // _BLK_S, b),
            in_specs=[
                pl.BlockSpec((1, _BLK_S, d), lambda j, i: (i, j, 0),
                             pipeline_mode=pl.Buffered(buffer_count=_NBUF)),
                pl.BlockSpec((_BLK_S, d), lambda j, i: (j, 0)),
            ],
            out_specs=[
                pl.BlockSpec((1, _BLK_S, d), lambda j, i: (i, j, 0)),
            ],
        )
        pipeline(x_hbm, t_hbm, o_hbm)

    return pl.pallas_call(
        outer,
        in_specs=[
            pl.BlockSpec(memory_space=pl.ANY),
            pl.BlockSpec(memory_space=pl.ANY),
        ],
        out_specs=pl.BlockSpec(memory_space=pl.ANY),
        out_shape=jax.ShapeDtypeStruct((b, s, d), x.dtype),
    )(x, table_s)
